# baseline (speedup 1.0000x reference)
import functools

import jax
import jax.numpy as jnp
from jax import lax
from jax.experimental import pallas as pl
from jax.experimental.pallas import tpu as pltpu
from jax.experimental.pallas import tpu_sc as plsc

_EPS = 1e-20
_R = 128
_C = 100000
_B = 8192
_NB = (_C + _B - 1) // _B

_BIG_F32 = 1e9

_NC = 2
_NS = 16
_ROWS_PER_TILE = _R // (_NC * _NS)
_CHUNK = 20000
_FILL_UNROLL = 10


def _argmax_body(x_ref, u_ref, idx_out, max_ref):
    j = pl.program_id(0)
    col0 = (j * _B).astype(jnp.float32)
    iota_f = lax.broadcasted_iota(jnp.int32, (_R, _B), 1).astype(jnp.float32)
    gcol = col0 + iota_f

    t = -jnp.log(u_ref[...] + _EPS) + _EPS
    f = jnp.exp(x_ref[...]) / t
    f = jnp.where(gcol < float(_C), f, -1.0)
    m = jnp.max(f, axis=1, keepdims=True)
    cand = jnp.min(jnp.where(f == m, gcol, _BIG_F32), axis=1, keepdims=True)

    @pl.when(j == 0)
    def _init():
        max_ref[...] = m
        idx_out[...] = cand.astype(jnp.int32)

    @pl.when(j > 0)
    def _acc():
        better = m > max_ref[...]
        max_ref[...] = jnp.where(better, m, max_ref[...])
        idx_out[...] = jnp.where(better, cand.astype(jnp.int32), idx_out[...])


def _tc_argmax(x, U):
    return pl.pallas_call(
        _argmax_body,
        grid=(_NB,),
        in_specs=[
            pl.BlockSpec((_R, _B), lambda j: (0, j)),
            pl.BlockSpec((_R, _B), lambda j: (0, j)),
        ],
        out_specs=pl.BlockSpec((_R, 1), lambda j: (0, 0)),
        out_shape=jax.ShapeDtypeStruct((_R, 1), jnp.int32),
        scratch_shapes=[pltpu.VMEM((_R, 1), jnp.float32)],
        compiler_params=pltpu.CompilerParams(
            dimension_semantics=("arbitrary",),
        ),
    )(x, U)


def _sc_zero_body(out_hbm, zero_v, sem):
    c = lax.axis_index("c")
    s = lax.axis_index("s")

    zeros16 = jnp.zeros((16,), jnp.float32)

    def fill(i, carry):
        base = i * (16 * _FILL_UNROLL)
        for u in range(_FILL_UNROLL):
            zero_v[pl.ds(base + u * 16, 16)] = zeros16
        return carry

    lax.fori_loop(0, _CHUNK // (16 * _FILL_UNROLL), fill, 0)

    row0 = (_NS * c + s) * _ROWS_PER_TILE
    copies = []
    for j in range(_ROWS_PER_TILE):
        for k in range(_C // _CHUNK):
            off = (row0 + j) * _C + k * _CHUNK
            copies.append(
                pltpu.make_async_copy(zero_v, out_hbm.at[pl.ds(off, _CHUNK)], sem)
            )
    for cp in copies:
        cp.start()
    for cp in copies:
        cp.wait()


@functools.lru_cache(maxsize=1)
def _sc_zero_call():
    return pl.kernel(
        _sc_zero_body,
        out_type=jax.ShapeDtypeStruct((_R * _C,), jnp.float32),
        mesh=plsc.VectorSubcoreMesh(
            core_axis_name="c", subcore_axis_name="s",
            num_cores=_NC, num_subcores=_NS,
        ),
        scratch_types=[
            pltpu.VMEM((_CHUNK,), jnp.float32),
            pltpu.SemaphoreType.DMA,
        ],
    )


def _scatter_body(idx_smem, idx_vmem, zeros_ref, out_ref, seg_v, sem):
    del zeros_ref
    lane = lax.broadcasted_iota(jnp.int32, (8, 128), 1)
    for r in range(_R):
        g = r // 8
        idx_col = idx_vmem[g * 8:(g + 1) * 8, :]
        cidx = idx_smem[r, 0]
        start = pl.multiple_of((cidx >> 7) << 7, 128)
        patch = ((lane + start) == idx_col).astype(jnp.float32)
        seg_v[r] = patch
    copies = []
    for r in range(_R):
        cidx = idx_smem[r, 0]
        start = pl.multiple_of((cidx >> 7) << 7, 128)
        copies.append(
            pltpu.make_async_copy(
                seg_v.at[r],
                out_ref.at[pl.ds((r // 8) * 8, 8), pl.ds(start, 128)],
                sem,
            )
        )
    for cp in copies:
        cp.start()
    for cp in copies:
        cp.wait()


def _tc_scatter(idx, zeros_flat):
    zeros2d = zeros_flat.reshape(_R, _C)
    return pl.pallas_call(
        _scatter_body,
        in_specs=[
            pl.BlockSpec(memory_space=pltpu.SMEM),
            pl.BlockSpec(memory_space=pltpu.VMEM),
            pl.BlockSpec(memory_space=pl.ANY),
        ],
        out_specs=pl.BlockSpec(memory_space=pl.ANY),
        out_shape=jax.ShapeDtypeStruct((_R, _C), jnp.float32),
        scratch_shapes=[
            pltpu.VMEM((_R, 8, 128), jnp.float32),
            pltpu.SemaphoreType.DMA,
        ],
        input_output_aliases={2: 0},
    )(idx, idx, zeros2d)


@jax.jit
def kernel(x, U):
    zeros_flat = _sc_zero_call()()
    idx = _tc_argmax(x, U)
    return _tc_scatter(idx, zeros_flat)


# speedup vs baseline: 1.4448x; 1.4448x over previous
import jax
import jax.numpy as jnp
from jax import lax
from jax.experimental import pallas as pl
from jax.experimental.pallas import tpu as pltpu

_EPS = 1e-20
_R = 128
_C = 100000
_RB = 8

_BIG_F32 = 1e9


def _body(x_ref, u_ref, out_ref):
    gcol = lax.broadcasted_iota(jnp.int32, (_RB, _C), 1).astype(jnp.float32)
    t = -jnp.log(u_ref[...] + _EPS) + _EPS
    f = jnp.exp(x_ref[...]) / t
    m = jnp.max(f, axis=1, keepdims=True)
    cand = jnp.min(jnp.where(f == m, gcol, _BIG_F32), axis=1, keepdims=True)
    out_ref[...] = (gcol == cand).astype(jnp.float32)


@jax.jit
def kernel(x, U):
    return pl.pallas_call(
        _body,
        grid=(_R // _RB,),
        in_specs=[
            pl.BlockSpec((_RB, _C), lambda j: (j, 0)),
            pl.BlockSpec((_RB, _C), lambda j: (j, 0)),
        ],
        out_specs=pl.BlockSpec((_RB, _C), lambda j: (j, 0)),
        out_shape=jax.ShapeDtypeStruct((_R, _C), jnp.float32),
        compiler_params=pltpu.CompilerParams(
            dimension_semantics=("arbitrary",),
        ),
    )(x, U)
